# baseline (speedup 1.0000x reference)
import functools

import jax
import jax.numpy as jnp
from jax import lax
from jax.experimental import pallas as pl
from jax.experimental.pallas import tpu as pltpu
from jax.experimental.pallas import tpu_sc as plsc

V = 1000
N_TOKENS = 16
HID = 256
B, S, L = 8, 4, 256
NW = 32
TPW = (B * S * L) // NW
ROWS_PER_SLAB = N_TOKENS + L
CHUNK = 64
MAIN_W = 896
TAIL_W = 128
TAIL_OFF_BLK = 7


def _learned_prompt(u, v, shared_prompt, mlp_w, mlp_b):
    def body(u_ref, v_ref, sp_ref, w_ref, b_ref, out_ref):
        learned = (u_ref[...] * v_ref[...]) * sp_ref[...]
        out_ref[...] = (
            jnp.dot(learned, w_ref[...], preferred_element_type=jnp.float32)
            + b_ref[...][None, :]
        )

    return pl.pallas_call(
        body,
        out_shape=jax.ShapeDtypeStruct((N_TOKENS, V), jnp.float32),
    )(u, v, shared_prompt, mlp_w, mlp_b)


def _sc_main(tokens_flat, wte, learned):
    mesh = plsc.VectorSubcoreMesh(core_axis_name="c", subcore_axis_name="s")
    n_chunks = TPW // CHUNK

    @functools.partial(
        pl.kernel,
        out_type=jax.ShapeDtypeStruct((B * S * ROWS_PER_SLAB, V), jnp.float32),
        mesh=mesh,
        scratch_types=[
            pltpu.VMEM((n_chunks, CHUNK), jnp.int32),
            pltpu.VMEM((CHUNK, MAIN_W), jnp.float32),
            pltpu.VMEM((CHUNK, MAIN_W), jnp.float32),
            pltpu.SemaphoreType.DMA,
            pltpu.SemaphoreType.DMA,
        ],
    )
    def k(tok_hbm, wte_hbm, learned_hbm, out_hbm, idx_v, rows0, rows1,
          sem0, sem1):
        wid = lax.axis_index("s") * 2 + lax.axis_index("c")
        out_base = wid * ROWS_PER_SLAB

        pltpu.sync_copy(tok_hbm.at[wid], idx_v)

        pltpu.sync_copy(learned_hbm, out_hbm.at[pl.ds(out_base, N_TOKENS)])

        bufs = (rows0, rows1)
        sems = (sem0, sem1)
        copies = []

        def scatter(c):
            pltpu.sync_copy(
                bufs[c % 2],
                out_hbm.at[pl.ds(out_base + N_TOKENS + c * CHUNK, CHUNK),
                           pl.ds(0, MAIN_W)],
            )

        for c in range(n_chunks):
            copies.append(
                pltpu.async_copy(wte_hbm.at[idx_v.at[c]], bufs[c % 2],
                                 sems[c % 2])
            )
            if c >= 1:
                copies[c - 1].wait()
                scatter(c - 1)
        copies[n_chunks - 1].wait()
        scatter(n_chunks - 1)

    return k(tokens_flat, wte, learned)


def _tc_tail(out, tokens_2d, wte_tail, learned_tail):
    def body(_, tok_ref, wte_ref, learned_ref, out_ref):
        toks = tok_ref[0, 0, :]
        onehot = (
            toks[:, None]
            == lax.broadcasted_iota(jnp.int32, (L, V), 1)
        ).astype(jnp.float32)
        gathered = jnp.dot(onehot, wte_ref[...],
                           preferred_element_type=jnp.float32)
        out_ref[0:N_TOKENS, :] = learned_ref[...]
        out_ref[N_TOKENS:ROWS_PER_SLAB, :] = gathered

    grid_spec = pltpu.PrefetchScalarGridSpec(
        num_scalar_prefetch=0,
        grid=(NW,),
        in_specs=[
            pl.BlockSpec(memory_space=pl.ANY),
            pl.BlockSpec((1, 1, L), lambda i: (i, 0, 0)),
            pl.BlockSpec((1000, TAIL_W), lambda i: (0, TAIL_OFF_BLK)),
            pl.BlockSpec((N_TOKENS, TAIL_W), lambda i: (0, TAIL_OFF_BLK)),
        ],
        out_specs=pl.BlockSpec((ROWS_PER_SLAB, TAIL_W),
                               lambda i: (i, TAIL_OFF_BLK)),
    )
    return pl.pallas_call(
        body,
        grid_spec=grid_spec,
        out_shape=jax.ShapeDtypeStruct((B * S * ROWS_PER_SLAB, V),
                                       jnp.float32),
        input_output_aliases={0: 0},
    )(out, tokens_2d, wte_tail, learned_tail)


def kernel(tokens, wte, mlp_w, mlp_b, shared_prompt, u, v):
    learned = _learned_prompt(u, v, shared_prompt, mlp_w, mlp_b)
    tokens_i32 = tokens.astype(jnp.int32)
    tokens_flat = tokens_i32.reshape(NW, TPW // CHUNK, CHUNK)
    out = _sc_main(tokens_flat, wte[:, :MAIN_W], learned)
    out = _tc_tail(
        out,
        tokens_i32.reshape(NW, 1, TPW),
        wte,
        learned,
    )
    return out.reshape(B, S, ROWS_PER_SLAB, V)


# speedup vs baseline: 1.5871x; 1.5871x over previous
import functools

import jax
import jax.numpy as jnp
from jax import lax
from jax.experimental import pallas as pl
from jax.experimental.pallas import tpu as pltpu
from jax.experimental.pallas import tpu_sc as plsc

V = 1000
VPAD = 1024
N_TOKENS = 16
HID = 256
B, S, L = 8, 4, 256
NW = 32
TPW = (B * S * L) // NW
ROWS_PER_SLAB = N_TOKENS + L
CHUNK = 32
MAIN_W = 896
TAIL_BLK = 7


def _learned_prompt(u, v, shared_prompt, mlp_w, mlp_b):
    def body(u_ref, v_ref, sp_ref, w_ref, b_ref, out_ref):
        learned = (u_ref[...] * v_ref[...]) * sp_ref[...]
        out_ref[...] = (
            jnp.dot(learned, w_ref[...], preferred_element_type=jnp.float32)
            + b_ref[...][None, :]
        )

    return pl.pallas_call(
        body,
        out_shape=jax.ShapeDtypeStruct((N_TOKENS, V), jnp.float32),
    )(u, v, shared_prompt, mlp_w, mlp_b)


def _sc_main(tokens_flat, wte_pad, learned):
    mesh = plsc.VectorSubcoreMesh(core_axis_name="c", subcore_axis_name="s")
    n_chunks = TPW // CHUNK

    @functools.partial(
        pl.kernel,
        out_type=(
            jax.ShapeDtypeStruct((B * S * ROWS_PER_SLAB, V), jnp.float32),
            jax.ShapeDtypeStruct((B * S * L, 128), jnp.float32),
        ),
        mesh=mesh,
        scratch_types=[
            pltpu.VMEM((n_chunks, CHUNK), jnp.int32),
            pltpu.VMEM((CHUNK, VPAD), jnp.float32),
            pltpu.VMEM((CHUNK, VPAD), jnp.float32),
            pltpu.VMEM((N_TOKENS, V), jnp.float32),
            pltpu.SemaphoreType.DMA,
            pltpu.SemaphoreType.DMA,
        ],
    )
    def k(tok_hbm, wte_hbm, learned_hbm, out_hbm, tail_hbm, idx_v,
          rows0, rows1, learned_v, sem0, sem1):
        wid = lax.axis_index("s") * 2 + lax.axis_index("c")
        out_base = wid * ROWS_PER_SLAB
        tail_base = wid * TPW

        pltpu.sync_copy(tok_hbm.at[wid], idx_v)

        pltpu.sync_copy(learned_hbm, learned_v)
        pltpu.sync_copy(learned_v, out_hbm.at[pl.ds(out_base, N_TOKENS)])

        def scatter(buf, c):
            for t in range(MAIN_W // 128):
                pltpu.sync_copy(
                    buf.at[:, pl.ds(t * 128, 128)],
                    out_hbm.at[pl.ds(out_base + N_TOKENS + c * CHUNK, CHUNK),
                               pl.ds(t * 128, 128)],
                )
            pltpu.sync_copy(
                buf.at[:, pl.ds(MAIN_W, 128)],
                tail_hbm.at[pl.ds(tail_base + c * CHUNK, CHUNK)],
            )

        bufs = (rows0, rows1)
        sems = (sem0, sem1)
        copies = []
        for c in range(n_chunks):
            copies.append(
                pltpu.async_copy(wte_hbm.at[idx_v.at[c]], bufs[c % 2],
                                 sems[c % 2])
            )
            if c >= 1:
                copies[c - 1].wait()
                scatter(bufs[(c - 1) % 2], c - 1)
        copies[n_chunks - 1].wait()
        scatter(bufs[(n_chunks - 1) % 2], n_chunks - 1)

    return k(tokens_flat, wte_pad, learned)


def _tc_tail(out, tail, learned):
    def body(_, tail_ref, learned_ref, out_ref):
        out_ref[0:N_TOKENS, :] = learned_ref[...]
        out_ref[N_TOKENS:ROWS_PER_SLAB, :] = tail_ref[...]

    grid_spec = pltpu.PrefetchScalarGridSpec(
        num_scalar_prefetch=0,
        grid=(NW,),
        in_specs=[
            pl.BlockSpec(memory_space=pl.ANY),
            pl.BlockSpec((L, 128), lambda i: (i, 0)),
            pl.BlockSpec((N_TOKENS, 128), lambda i: (0, TAIL_BLK)),
        ],
        out_specs=pl.BlockSpec((ROWS_PER_SLAB, 128),
                               lambda i: (i, TAIL_BLK)),
    )
    return pl.pallas_call(
        body,
        grid_spec=grid_spec,
        out_shape=jax.ShapeDtypeStruct((B * S * ROWS_PER_SLAB, V),
                                       jnp.float32),
        input_output_aliases={0: 0},
    )(out, tail, learned)


def kernel(tokens, wte, mlp_w, mlp_b, shared_prompt, u, v):
    learned = _learned_prompt(u, v, shared_prompt, mlp_w, mlp_b)
    tokens_flat = tokens.reshape(NW, TPW // CHUNK, CHUNK).astype(jnp.int32)
    wte_pad = jnp.pad(wte, ((0, 0), (0, VPAD - V)))
    out, tail = _sc_main(tokens_flat, wte_pad, learned)
    out = _tc_tail(out, tail, learned)
    return out.reshape(B, S, ROWS_PER_SLAB, V)


# speedup vs baseline: 1.8450x; 1.1625x over previous
import functools

import jax
import jax.numpy as jnp
from jax import lax
from jax.experimental import pallas as pl
from jax.experimental.pallas import tpu as pltpu
from jax.experimental.pallas import tpu_sc as plsc

V = 1000
VPAD = 1024
N_TOKENS = 16
HID = 256
B, S, L = 8, 4, 256
NW = 32
TPW = (B * S * L) // NW
ROWS_PER_SLAB = N_TOKENS + L
CHUNK = 32
MAIN_W = 896
TAIL_BLK = 7


def _learned_prompt(u, v, shared_prompt, mlp_w, mlp_b):
    def body(u_ref, v_ref, sp_ref, w_ref, b_ref, out_ref):
        learned = (u_ref[...] * v_ref[...]) * sp_ref[...]
        out_ref[...] = (
            jnp.dot(learned, w_ref[...], preferred_element_type=jnp.float32)
            + b_ref[...][None, :]
        )

    return pl.pallas_call(
        body,
        out_shape=jax.ShapeDtypeStruct((N_TOKENS, V), jnp.float32),
    )(u, v, shared_prompt, mlp_w, mlp_b)


def _sc_main(tokens_flat, wte_pad, learned):
    mesh = plsc.VectorSubcoreMesh(core_axis_name="c", subcore_axis_name="s")
    n_chunks = TPW // CHUNK

    @functools.partial(
        pl.kernel,
        out_type=(
            jax.ShapeDtypeStruct((B * S * ROWS_PER_SLAB, V), jnp.float32),
            jax.ShapeDtypeStruct((B * S * L, 128), jnp.float32),
        ),
        mesh=mesh,
        scratch_types=[
            pltpu.VMEM((n_chunks, CHUNK), jnp.int32),
            pltpu.VMEM((CHUNK, VPAD), jnp.float32),
            pltpu.VMEM((CHUNK, VPAD), jnp.float32),
            pltpu.VMEM((N_TOKENS, V), jnp.float32),
            pltpu.SemaphoreType.DMA,
            pltpu.SemaphoreType.DMA,
        ],
    )
    def k(tok_hbm, wte_hbm, learned_hbm, out_hbm, tail_hbm, idx_v,
          rows0, rows1, learned_v, sem0, sem1):
        wid = lax.axis_index("s") * 2 + lax.axis_index("c")
        out_base = wid * ROWS_PER_SLAB
        tail_base = wid * TPW

        pltpu.sync_copy(tok_hbm.at[wid], idx_v)

        pltpu.sync_copy(learned_hbm, learned_v)
        pltpu.sync_copy(learned_v, out_hbm.at[pl.ds(out_base, N_TOKENS)])

        def scatter(buf, c):
            for t in range(MAIN_W // 128):
                pltpu.sync_copy(
                    buf.at[:, pl.ds(t * 128, 128)],
                    out_hbm.at[pl.ds(out_base + N_TOKENS + c * CHUNK, CHUNK),
                               pl.ds(t * 128, 128)],
                )
            pltpu.sync_copy(
                buf.at[:, pl.ds(MAIN_W, 128)],
                tail_hbm.at[pl.ds(tail_base + c * CHUNK, CHUNK)],
            )

        bufs = (rows0, rows1)
        sems = (sem0, sem1)
        copies = []
        for c in range(n_chunks):
            copies.append(
                pltpu.async_copy(wte_hbm.at[idx_v.at[c]], bufs[c % 2],
                                 sems[c % 2])
            )
            if c >= 1:
                copies[c - 1].wait()
                scatter(bufs[(c - 1) % 2], c - 1)
        copies[n_chunks - 1].wait()
        scatter(bufs[(n_chunks - 1) % 2], n_chunks - 1)

    return k(tokens_flat, wte_pad, learned)


def _tc_tail(out, tail, learned):
    SLABS = 4

    def body(_, tail_ref, learned_ref, out_ref):
        for s in range(SLABS):
            out_ref[s * ROWS_PER_SLAB:s * ROWS_PER_SLAB + N_TOKENS, :] = (
                learned_ref[...]
            )
            out_ref[s * ROWS_PER_SLAB + N_TOKENS:(s + 1) * ROWS_PER_SLAB,
                    :] = tail_ref[s * L:(s + 1) * L, :]

    grid_spec = pltpu.PrefetchScalarGridSpec(
        num_scalar_prefetch=0,
        grid=(NW // SLABS,),
        in_specs=[
            pl.BlockSpec(memory_space=pl.ANY),
            pl.BlockSpec((SLABS * L, 128), lambda i: (i, 0)),
            pl.BlockSpec((N_TOKENS, 128), lambda i: (0, TAIL_BLK)),
        ],
        out_specs=pl.BlockSpec((SLABS * ROWS_PER_SLAB, 128),
                               lambda i: (i, TAIL_BLK)),
    )
    return pl.pallas_call(
        body,
        grid_spec=grid_spec,
        out_shape=jax.ShapeDtypeStruct((B * S * ROWS_PER_SLAB, V),
                                       jnp.float32),
        input_output_aliases={0: 0},
    )(out, tail, learned)


def kernel(tokens, wte, mlp_w, mlp_b, shared_prompt, u, v):
    learned = _learned_prompt(u, v, shared_prompt, mlp_w, mlp_b)
    tokens_flat = tokens.reshape(NW, TPW // CHUNK, CHUNK).astype(jnp.int32)
    wte_pad = jnp.pad(wte, ((0, 0), (0, VPAD - V)))
    out, tail = _sc_main(tokens_flat, wte_pad, learned)
    out = _tc_tail(out, tail, learned)
    return out.reshape(B, S, ROWS_PER_SLAB, V)


# speedup vs baseline: 1.9350x; 1.0488x over previous
import functools

import jax
import jax.numpy as jnp
from jax import lax
from jax.experimental import pallas as pl
from jax.experimental.pallas import tpu as pltpu
from jax.experimental.pallas import tpu_sc as plsc

V = 1000
VPAD = 1024
N_TOKENS = 16
HID = 256
B, S, L = 8, 4, 256
NW = 32
TPW = (B * S * L) // NW
ROWS_PER_SLAB = N_TOKENS + L
CHUNK = 32
MAIN_W = 896
TAIL_BLK = 7


def _learned_prompt(u, v, shared_prompt, mlp_w, mlp_b):
    def body(u_ref, v_ref, sp_ref, w_ref, b_ref, out_ref):
        learned = (u_ref[...] * v_ref[...]) * sp_ref[...]
        out_ref[...] = (
            jnp.dot(learned, w_ref[...], preferred_element_type=jnp.float32)
            + b_ref[...][None, :]
        )

    return pl.pallas_call(
        body,
        out_shape=jax.ShapeDtypeStruct((N_TOKENS, V), jnp.float32),
    )(u, v, shared_prompt, mlp_w, mlp_b)


def _sc_main(tokens_flat, wte_pad, learned):
    mesh = plsc.VectorSubcoreMesh(core_axis_name="c", subcore_axis_name="s")
    n_chunks = TPW // CHUNK

    @functools.partial(
        pl.kernel,
        out_type=(
            jax.ShapeDtypeStruct((B * S * ROWS_PER_SLAB, V), jnp.float32),
            jax.ShapeDtypeStruct((B * S * L, 128), jnp.float32),
        ),
        mesh=mesh,
        scratch_types=[
            pltpu.VMEM((n_chunks, CHUNK), jnp.int32),
            pltpu.VMEM((CHUNK, VPAD), jnp.float32),
            pltpu.VMEM((CHUNK, VPAD), jnp.float32),
            pltpu.VMEM((N_TOKENS, V), jnp.float32),
            pltpu.SemaphoreType.DMA,
            pltpu.SemaphoreType.DMA,
            pltpu.SemaphoreType.DMA,
            pltpu.SemaphoreType.DMA,
        ],
    )
    def k(tok_hbm, wte_hbm, learned_hbm, out_hbm, tail_hbm, idx_v,
          rows0, rows1, learned_v, sem0, sem1, ssem0, ssem1):
        wid = lax.axis_index("s") * 2 + lax.axis_index("c")
        out_base = wid * ROWS_PER_SLAB
        tail_base = wid * TPW

        pltpu.sync_copy(tok_hbm.at[wid], idx_v)

        pltpu.sync_copy(learned_hbm, learned_v)
        pltpu.sync_copy(learned_v, out_hbm.at[pl.ds(out_base, N_TOKENS)])

        def scatter(buf, c, ssem):
            handles = []
            for t in range(MAIN_W // 128):
                handles.append(pltpu.async_copy(
                    buf.at[:, pl.ds(t * 128, 128)],
                    out_hbm.at[pl.ds(out_base + N_TOKENS + c * CHUNK, CHUNK),
                               pl.ds(t * 128, 128)],
                    ssem,
                ))
            handles.append(pltpu.async_copy(
                buf.at[:, pl.ds(MAIN_W, 128)],
                tail_hbm.at[pl.ds(tail_base + c * CHUNK, CHUNK)],
                ssem,
            ))
            return handles

        bufs = (rows0, rows1)
        gsems = (sem0, sem1)
        ssems = (ssem0, ssem1)
        gathers = []
        scatters = {}
        for c in range(n_chunks):
            if c >= 2:
                for h in scatters[c - 2]:
                    h.wait()
            gathers.append(
                pltpu.async_copy(wte_hbm.at[idx_v.at[c]], bufs[c % 2],
                                 gsems[c % 2])
            )
            if c >= 1:
                gathers[c - 1].wait()
                scatters[c - 1] = scatter(bufs[(c - 1) % 2], c - 1,
                                          ssems[(c - 1) % 2])
        gathers[n_chunks - 1].wait()
        scatters[n_chunks - 1] = scatter(bufs[(n_chunks - 1) % 2],
                                         n_chunks - 1,
                                         ssems[(n_chunks - 1) % 2])
        for c in (n_chunks - 2, n_chunks - 1):
            for h in scatters[c]:
                h.wait()

    return k(tokens_flat, wte_pad, learned)


def _tc_tail(out, tail, learned):
    SLABS = 8

    def body(_, tail_ref, learned_ref, out_ref):
        for s in range(SLABS):
            out_ref[s * ROWS_PER_SLAB:s * ROWS_PER_SLAB + N_TOKENS, :] = (
                learned_ref[...]
            )
            out_ref[s * ROWS_PER_SLAB + N_TOKENS:(s + 1) * ROWS_PER_SLAB,
                    :] = tail_ref[s * L:(s + 1) * L, :]

    grid_spec = pltpu.PrefetchScalarGridSpec(
        num_scalar_prefetch=0,
        grid=(NW // SLABS,),
        in_specs=[
            pl.BlockSpec(memory_space=pl.ANY),
            pl.BlockSpec((SLABS * L, 128), lambda i: (i, 0)),
            pl.BlockSpec((N_TOKENS, 128), lambda i: (0, TAIL_BLK)),
        ],
        out_specs=pl.BlockSpec((SLABS * ROWS_PER_SLAB, 128),
                               lambda i: (i, TAIL_BLK)),
    )
    return pl.pallas_call(
        body,
        grid_spec=grid_spec,
        out_shape=jax.ShapeDtypeStruct((B * S * ROWS_PER_SLAB, V),
                                       jnp.float32),
        input_output_aliases={0: 0},
    )(out, tail, learned)


def kernel(tokens, wte, mlp_w, mlp_b, shared_prompt, u, v):
    learned = _learned_prompt(u, v, shared_prompt, mlp_w, mlp_b)
    tokens_flat = tokens.reshape(NW, TPW // CHUNK, CHUNK).astype(jnp.int32)
    wte_pad = jnp.pad(wte, ((0, 0), (0, VPAD - V)))
    out, tail = _sc_main(tokens_flat, wte_pad, learned)
    out = _tc_tail(out, tail, learned)
    return out.reshape(B, S, ROWS_PER_SLAB, V)


# speedup vs baseline: 1.9512x; 1.0083x over previous
import functools

import jax
import jax.numpy as jnp
from jax import lax
from jax.experimental import pallas as pl
from jax.experimental.pallas import tpu as pltpu
from jax.experimental.pallas import tpu_sc as plsc

V = 1000
VPAD = 1024
N_TOKENS = 16
HID = 256
B, S, L = 8, 4, 256
NW = 32
TPW = (B * S * L) // NW
ROWS_PER_SLAB = N_TOKENS + L
CHUNK = 32
MAIN_W = 896
TAIL_BLK = 7


def _learned_prompt(u, v, shared_prompt, mlp_w, mlp_b):
    def body(u_ref, v_ref, sp_ref, w_ref, b_ref, out_ref):
        learned = (u_ref[...] * v_ref[...]) * sp_ref[...]
        out_ref[...] = (
            jnp.dot(learned, w_ref[...], preferred_element_type=jnp.float32)
            + b_ref[...][None, :]
        )

    return pl.pallas_call(
        body,
        out_shape=jax.ShapeDtypeStruct((N_TOKENS, V), jnp.float32),
    )(u, v, shared_prompt, mlp_w, mlp_b)


def _sc_main(tokens_flat, wte_pad, learned):
    mesh = plsc.VectorSubcoreMesh(core_axis_name="c", subcore_axis_name="s")
    n_chunks = TPW // CHUNK

    @functools.partial(
        pl.kernel,
        out_type=(
            jax.ShapeDtypeStruct((B * S * ROWS_PER_SLAB, V), jnp.float32),
            jax.ShapeDtypeStruct((B * S * L, 128), jnp.float32),
        ),
        mesh=mesh,
        scratch_types=[
            pltpu.VMEM((n_chunks, CHUNK), jnp.int32),
            pltpu.VMEM((CHUNK, VPAD), jnp.float32),
            pltpu.VMEM((CHUNK, VPAD), jnp.float32),
            pltpu.VMEM((CHUNK, VPAD), jnp.float32),
            pltpu.VMEM((N_TOKENS, V), jnp.float32),
            pltpu.SemaphoreType.DMA,
            pltpu.SemaphoreType.DMA,
            pltpu.SemaphoreType.DMA,
            pltpu.SemaphoreType.DMA,
            pltpu.SemaphoreType.DMA,
            pltpu.SemaphoreType.DMA,
        ],
    )
    def k(tok_hbm, wte_hbm, learned_hbm, out_hbm, tail_hbm, idx_v,
          rows0, rows1, rows2, learned_v, sem0, sem1, sem2,
          ssem0, ssem1, ssem2):
        wid = lax.axis_index("s") * 2 + lax.axis_index("c")
        out_base = wid * ROWS_PER_SLAB
        tail_base = wid * TPW

        pltpu.sync_copy(tok_hbm.at[wid], idx_v)

        pltpu.sync_copy(learned_hbm, learned_v)
        pltpu.sync_copy(learned_v, out_hbm.at[pl.ds(out_base, N_TOKENS)])

        def scatter(buf, c, ssem):
            handles = []
            for t in range(MAIN_W // 128):
                handles.append(pltpu.async_copy(
                    buf.at[:, pl.ds(t * 128, 128)],
                    out_hbm.at[pl.ds(out_base + N_TOKENS + c * CHUNK, CHUNK),
                               pl.ds(t * 128, 128)],
                    ssem,
                ))
            handles.append(pltpu.async_copy(
                buf.at[:, pl.ds(MAIN_W, 128)],
                tail_hbm.at[pl.ds(tail_base + c * CHUNK, CHUNK)],
                ssem,
            ))
            return handles

        bufs = (rows0, rows1, rows2)
        gsems = (sem0, sem1, sem2)
        ssems = (ssem0, ssem1, ssem2)

        def gather(c):
            return pltpu.async_copy(wte_hbm.at[idx_v.at[c]], bufs[c % 3],
                                    gsems[c % 3])

        gathers = {0: gather(0), 1: gather(1)}
        scatters = {}
        for c in range(n_chunks):
            gathers[c].wait()
            scatters[c] = scatter(bufs[c % 3], c, ssems[c % 3])
            nxt = c + 2
            if nxt < n_chunks:
                if nxt - 3 >= 0:
                    for h in scatters[nxt - 3]:
                        h.wait()
                gathers[nxt] = gather(nxt)
        for c in range(max(0, n_chunks - 3), n_chunks):
            for h in scatters[c]:
                h.wait()

    return k(tokens_flat, wte_pad, learned)


def _tc_tail(out, tail, learned):
    SLABS = 8

    def body(_, tail_ref, learned_ref, out_ref):
        for s in range(SLABS):
            out_ref[s * ROWS_PER_SLAB:s * ROWS_PER_SLAB + N_TOKENS, :] = (
                learned_ref[...]
            )
            out_ref[s * ROWS_PER_SLAB + N_TOKENS:(s + 1) * ROWS_PER_SLAB,
                    :] = tail_ref[s * L:(s + 1) * L, :]

    grid_spec = pltpu.PrefetchScalarGridSpec(
        num_scalar_prefetch=0,
        grid=(NW // SLABS,),
        in_specs=[
            pl.BlockSpec(memory_space=pl.ANY),
            pl.BlockSpec((SLABS * L, 128), lambda i: (i, 0)),
            pl.BlockSpec((N_TOKENS, 128), lambda i: (0, TAIL_BLK)),
        ],
        out_specs=pl.BlockSpec((SLABS * ROWS_PER_SLAB, 128),
                               lambda i: (i, TAIL_BLK)),
    )
    return pl.pallas_call(
        body,
        grid_spec=grid_spec,
        out_shape=jax.ShapeDtypeStruct((B * S * ROWS_PER_SLAB, V),
                                       jnp.float32),
        input_output_aliases={0: 0},
    )(out, tail, learned)


def kernel(tokens, wte, mlp_w, mlp_b, shared_prompt, u, v):
    learned = _learned_prompt(u, v, shared_prompt, mlp_w, mlp_b)
    tokens_flat = tokens.reshape(NW, TPW // CHUNK, CHUNK).astype(jnp.int32)
    wte_pad = jnp.pad(wte, ((0, 0), (0, VPAD - V)))
    out, tail = _sc_main(tokens_flat, wte_pad, learned)
    out = _tc_tail(out, tail, learned)
    return out.reshape(B, S, ROWS_PER_SLAB, V)


# speedup vs baseline: 2.0599x; 1.0557x over previous
import functools

import jax
import jax.numpy as jnp
from jax import lax
from jax.experimental import pallas as pl
from jax.experimental.pallas import tpu as pltpu
from jax.experimental.pallas import tpu_sc as plsc

V = 1000
N_TOKENS = 16
HID = 256
B, S, L = 8, 4, 256
NW = 32
TPW = (B * S * L) // NW
ROWS_PER_SLAB = N_TOKENS + L
CHUNK = 32
MAIN_W = 896
TAIL_BLK = 7


def _learned_prompt(u, v, shared_prompt, mlp_w, mlp_b):
    def body(u_ref, v_ref, sp_ref, w_ref, b_ref, out_ref):
        learned = (u_ref[...] * v_ref[...]) * sp_ref[...]
        out_ref[...] = (
            jnp.dot(learned, w_ref[...], preferred_element_type=jnp.float32)
            + b_ref[...][None, :]
        )

    return pl.pallas_call(
        body,
        out_shape=jax.ShapeDtypeStruct((N_TOKENS, V), jnp.float32),
    )(u, v, shared_prompt, mlp_w, mlp_b)


def _sc_main(tokens_flat, wte_main, wte_tail, learned):
    mesh = plsc.VectorSubcoreMesh(core_axis_name="c", subcore_axis_name="s")
    n_chunks = TPW // CHUNK

    @functools.partial(
        pl.kernel,
        out_type=(
            jax.ShapeDtypeStruct((B * S * ROWS_PER_SLAB, V), jnp.float32),
            jax.ShapeDtypeStruct((B * S * L, 128), jnp.float32),
        ),
        mesh=mesh,
        scratch_types=[
            pltpu.VMEM((n_chunks, CHUNK), jnp.int32),
            pltpu.VMEM((CHUNK, MAIN_W), jnp.float32),
            pltpu.VMEM((CHUNK, MAIN_W), jnp.float32),
            pltpu.VMEM((CHUNK, MAIN_W), jnp.float32),
            pltpu.VMEM((CHUNK, 128), jnp.float32),
            pltpu.VMEM((CHUNK, 128), jnp.float32),
            pltpu.VMEM((CHUNK, 128), jnp.float32),
            pltpu.VMEM((N_TOKENS, V), jnp.float32),
            pltpu.SemaphoreType.DMA,
            pltpu.SemaphoreType.DMA,
            pltpu.SemaphoreType.DMA,
            pltpu.SemaphoreType.DMA,
            pltpu.SemaphoreType.DMA,
            pltpu.SemaphoreType.DMA,
        ],
    )
    def k(tok_hbm, wmain_hbm, wtail_hbm, learned_hbm, out_hbm, tail_hbm,
          idx_v, m0, m1, m2, t0, t1, t2, learned_v,
          gs0, gs1, gs2, ss0, ss1, ss2):
        wid = lax.axis_index("s") * 2 + lax.axis_index("c")
        out_base = wid * ROWS_PER_SLAB
        tail_base = wid * TPW

        pltpu.sync_copy(tok_hbm.at[wid], idx_v)

        pltpu.sync_copy(learned_hbm, learned_v)
        pltpu.sync_copy(learned_v, out_hbm.at[pl.ds(out_base, N_TOKENS)])

        mbufs = (m0, m1, m2)
        tbufs = (t0, t1, t2)
        gsems = (gs0, gs1, gs2)
        ssems = (ss0, ss1, ss2)

        def gather(c):
            return (
                pltpu.async_copy(wmain_hbm.at[idx_v.at[c]], mbufs[c % 3],
                                 gsems[c % 3]),
                pltpu.async_copy(wtail_hbm.at[idx_v.at[c]], tbufs[c % 3],
                                 gsems[c % 3]),
            )

        def scatter(c):
            return (
                pltpu.async_copy(
                    mbufs[c % 3],
                    out_hbm.at[pl.ds(out_base + N_TOKENS + c * CHUNK, CHUNK),
                               pl.ds(0, MAIN_W)],
                    ssems[c % 3],
                ),
                pltpu.async_copy(
                    tbufs[c % 3],
                    tail_hbm.at[pl.ds(tail_base + c * CHUNK, CHUNK)],
                    ssems[c % 3],
                ),
            )

        gathers = {0: gather(0), 1: gather(1)}
        scatters = {}
        for c in range(n_chunks):
            for h in gathers[c]:
                h.wait()
            scatters[c] = scatter(c)
            nxt = c + 2
            if nxt < n_chunks:
                if nxt - 3 >= 0:
                    for h in scatters[nxt - 3]:
                        h.wait()
                gathers[nxt] = gather(nxt)
        for c in range(max(0, n_chunks - 3), n_chunks):
            for h in scatters[c]:
                h.wait()

    return k(tokens_flat, wte_main, wte_tail, learned)


def _tc_tail(out, tail, learned):
    SLABS = 8

    def body(_, tail_ref, learned_ref, out_ref):
        for s in range(SLABS):
            out_ref[s * ROWS_PER_SLAB:s * ROWS_PER_SLAB + N_TOKENS, :] = (
                learned_ref[...]
            )
            out_ref[s * ROWS_PER_SLAB + N_TOKENS:(s + 1) * ROWS_PER_SLAB,
                    :] = tail_ref[s * L:(s + 1) * L, :]

    grid_spec = pltpu.PrefetchScalarGridSpec(
        num_scalar_prefetch=0,
        grid=(NW // SLABS,),
        in_specs=[
            pl.BlockSpec(memory_space=pl.ANY),
            pl.BlockSpec((SLABS * L, 128), lambda i: (i, 0)),
            pl.BlockSpec((N_TOKENS, 128), lambda i: (0, TAIL_BLK)),
        ],
        out_specs=pl.BlockSpec((SLABS * ROWS_PER_SLAB, 128),
                               lambda i: (i, TAIL_BLK)),
    )
    return pl.pallas_call(
        body,
        grid_spec=grid_spec,
        out_shape=jax.ShapeDtypeStruct((B * S * ROWS_PER_SLAB, V),
                                       jnp.float32),
        input_output_aliases={0: 0},
    )(out, tail, learned)


def kernel(tokens, wte, mlp_w, mlp_b, shared_prompt, u, v):
    learned = _learned_prompt(u, v, shared_prompt, mlp_w, mlp_b)
    tokens_flat = tokens.reshape(NW, TPW // CHUNK, CHUNK).astype(jnp.int32)
    wte_main = wte[:, :MAIN_W]
    wte_tail = jnp.pad(wte[:, MAIN_W:], ((0, 0), (0, 128 - (V - MAIN_W))))
    out, tail = _sc_main(tokens_flat, wte_main, wte_tail, learned)
    out = _tc_tail(out, tail, learned)
    return out.reshape(B, S, ROWS_PER_SLAB, V)


# speedup vs baseline: 2.0654x; 1.0027x over previous
import functools

import jax
import jax.numpy as jnp
from jax import lax
from jax.experimental import pallas as pl
from jax.experimental.pallas import tpu as pltpu
from jax.experimental.pallas import tpu_sc as plsc

V = 1000
N_TOKENS = 16
HID = 256
B, S, L = 8, 4, 256
NW = 32
TPW = (B * S * L) // NW
ROWS_PER_SLAB = N_TOKENS + L
CHUNK = 32
MAIN_W = 896
TAIL_BLK = 7


def _learned_prompt(u, v, shared_prompt, mlp_w, mlp_b):
    def body(u_ref, v_ref, sp_ref, w_ref, b_ref, out_ref):
        learned = (u_ref[...] * v_ref[...]) * sp_ref[...]
        out_ref[...] = (
            jnp.dot(learned, w_ref[...], preferred_element_type=jnp.float32)
            + b_ref[...][None, :]
        )

    return pl.pallas_call(
        body,
        out_shape=jax.ShapeDtypeStruct((N_TOKENS, V), jnp.float32),
    )(u, v, shared_prompt, mlp_w, mlp_b)


def _sc_main(tokens_flat, wte_main, wte_tail, learned):
    mesh = plsc.VectorSubcoreMesh(core_axis_name="c", subcore_axis_name="s")
    n_chunks = TPW // CHUNK

    @functools.partial(
        pl.kernel,
        out_type=(
            jax.ShapeDtypeStruct((B * S * ROWS_PER_SLAB, V), jnp.float32),
            jax.ShapeDtypeStruct((B * S * L, 128), jnp.float32),
        ),
        mesh=mesh,
        scratch_types=[
            pltpu.VMEM((n_chunks, CHUNK), jnp.int32),
            pltpu.VMEM((CHUNK, MAIN_W), jnp.float32),
            pltpu.VMEM((CHUNK, MAIN_W), jnp.float32),
            pltpu.VMEM((CHUNK, MAIN_W), jnp.float32),
            pltpu.VMEM((CHUNK, 128), jnp.float32),
            pltpu.VMEM((CHUNK, 128), jnp.float32),
            pltpu.VMEM((CHUNK, 128), jnp.float32),
            pltpu.VMEM((N_TOKENS, V), jnp.float32),
            pltpu.SemaphoreType.DMA,
            pltpu.SemaphoreType.DMA,
            pltpu.SemaphoreType.DMA,
            pltpu.SemaphoreType.DMA,
            pltpu.SemaphoreType.DMA,
            pltpu.SemaphoreType.DMA,
        ],
    )
    def k(tok_hbm, wmain_hbm, wtail_hbm, learned_hbm, out_hbm, tail_hbm,
          idx_v, m0, m1, m2, t0, t1, t2, learned_v,
          gs0, gs1, gs2, ss0, ss1, ss2):
        wid = lax.axis_index("s") * 2 + lax.axis_index("c")
        out_base = wid * ROWS_PER_SLAB
        tail_base = wid * TPW

        pltpu.sync_copy(tok_hbm.at[wid], idx_v)

        mbufs = (m0, m1, m2)
        tbufs = (t0, t1, t2)
        gsems = (gs0, gs1, gs2)
        ssems = (ss0, ss1, ss2)

        def gather(c):
            return (
                pltpu.async_copy(wmain_hbm.at[idx_v.at[c]], mbufs[c % 3],
                                 gsems[c % 3]),
                pltpu.async_copy(wtail_hbm.at[idx_v.at[c]], tbufs[c % 3],
                                 gsems[c % 3]),
            )

        def scatter(c):
            return (
                pltpu.async_copy(
                    mbufs[c % 3],
                    out_hbm.at[pl.ds(out_base + N_TOKENS + c * CHUNK, CHUNK),
                               pl.ds(0, MAIN_W)],
                    ssems[c % 3],
                ),
                pltpu.async_copy(
                    tbufs[c % 3],
                    tail_hbm.at[pl.ds(tail_base + c * CHUNK, CHUNK)],
                    ssems[c % 3],
                ),
            )

        gathers = {0: gather(0), 1: gather(1)}

        pltpu.sync_copy(learned_hbm, learned_v)
        pltpu.sync_copy(learned_v, out_hbm.at[pl.ds(out_base, N_TOKENS)])

        scatters = {}
        for c in range(n_chunks):
            for h in gathers[c]:
                h.wait()
            scatters[c] = scatter(c)
            nxt = c + 2
            if nxt < n_chunks:
                if nxt - 3 >= 0:
                    for h in scatters[nxt - 3]:
                        h.wait()
                gathers[nxt] = gather(nxt)
        for c in range(max(0, n_chunks - 3), n_chunks):
            for h in scatters[c]:
                h.wait()

    return k(tokens_flat, wte_main, wte_tail, learned)


def _tc_tail(out, tail, learned):
    SLABS = 8

    def body(_, tail_ref, learned_ref, out_ref):
        for s in range(SLABS):
            out_ref[s * ROWS_PER_SLAB:s * ROWS_PER_SLAB + N_TOKENS, :] = (
                learned_ref[...]
            )
            out_ref[s * ROWS_PER_SLAB + N_TOKENS:(s + 1) * ROWS_PER_SLAB,
                    :] = tail_ref[s * L:(s + 1) * L, :]

    grid_spec = pltpu.PrefetchScalarGridSpec(
        num_scalar_prefetch=0,
        grid=(NW // SLABS,),
        in_specs=[
            pl.BlockSpec(memory_space=pl.ANY),
            pl.BlockSpec((SLABS * L, 128), lambda i: (i, 0)),
            pl.BlockSpec((N_TOKENS, 128), lambda i: (0, TAIL_BLK)),
        ],
        out_specs=pl.BlockSpec((SLABS * ROWS_PER_SLAB, 128),
                               lambda i: (i, TAIL_BLK)),
    )
    return pl.pallas_call(
        body,
        grid_spec=grid_spec,
        out_shape=jax.ShapeDtypeStruct((B * S * ROWS_PER_SLAB, V),
                                       jnp.float32),
        input_output_aliases={0: 0},
    )(out, tail, learned)


def kernel(tokens, wte, mlp_w, mlp_b, shared_prompt, u, v):
    learned = _learned_prompt(u, v, shared_prompt, mlp_w, mlp_b)
    tokens_flat = tokens.reshape(NW, TPW // CHUNK, CHUNK).astype(jnp.int32)
    wte_main = wte[:, :MAIN_W]
    wte_tail = jnp.pad(wte[:, MAIN_W:], ((0, 0), (0, 128 - (V - MAIN_W))))
    out, tail = _sc_main(tokens_flat, wte_main, wte_tail, learned)
    out = _tc_tail(out, tail, learned)
    return out.reshape(B, S, ROWS_PER_SLAB, V)
